# baseline (speedup 1.0000x reference)
import functools

import jax
import jax.numpy as jnp
from jax import lax
from jax.experimental import pallas as pl
from jax.experimental.pallas import tpu as pltpu
from jax.experimental.pallas import tpu_sc as plsc

BATCH = 16384
HIST = 50
EMBED_D = 32
NC, NS = 2, 16
NW = NC * NS
PW = BATCH // NW
HB = 5
BC = 128
NSTEP = (PW // BC) * (HIST // HB)


def _gather_body(xt_hbm, table_hbm, out_hbm,
                 idx_t, raw0, raw1, tb0, tb1, sg0, sg1, ss0, ss1):
    wid = lax.axis_index("s") * NC + lax.axis_index("c")
    raws = (raw0, raw1)
    tbs = (tb0, tb1)
    sgs = (sg0, sg1)
    sss = (ss0, ss1)

    pltpu.sync_copy(xt_hbm.at[wid], idx_t)

    iota = jax.lax.broadcasted_iota(jnp.int32, (16,), 0)
    bvecs = [iota + (16 * k) for k in range(8)]
    dcols = [jnp.full((16,), d, jnp.int32) for d in range(EMBED_D)]

    def step_coords(s):
        bc = s // (HIST // HB)
        h0 = (s % (HIST // HB)) * HB
        return bc, h0

    def issue(s, b):
        bc, h0 = step_coords(s)
        for hh in range(HB):
            pltpu.async_copy(
                table_hbm.at[idx_t.at[h0 + hh, pl.ds(bc * BC, BC)]],
                raws[b].at[pl.ds(hh * BC, BC)],
                sgs[b],
            )

    def drain(s, b):
        bc, h0 = step_coords(s)
        for hh in range(HB):
            pltpu.make_async_copy(
                table_hbm.at[idx_t.at[h0 + hh, pl.ds(bc * BC, BC)]],
                raws[b].at[pl.ds(hh * BC, BC)],
                sgs[b],
            ).wait()

    issue(0, 0)
    issue(1, 1)

    def body(i, carry):
        for b in (0, 1):
            s = 2 * i + b
            bc, h0 = step_coords(s)
            bt = 4 * wid + bc
            drain(s, b)

            @pl.when(s >= 2)
            def _():
                pltpu.make_async_copy(
                    tbs[b], out_hbm.at[pl.ds(h0, HB), :, bt], sss[b]
                ).wait()

            for hh in range(HB):
                rowvecs = [bv + (hh * BC) for bv in bvecs]
                for d in range(EMBED_D):
                    dcol = dcols[d]
                    dt = d // 8
                    dr = d % 8
                    for k in range(8):
                        v = plsc.load_gather(raws[b], [rowvecs[k], dcol])
                        tbs[b][hh, dt, dr, pl.ds(k * 16, 16)] = v

            pltpu.async_copy(tbs[b], out_hbm.at[pl.ds(h0, HB), :, bt], sss[b])

            nxt = s + 2

            @pl.when(nxt < NSTEP)
            def _():
                issue(nxt, b)

        return carry

    lax.fori_loop(0, NSTEP // 2, body, 0)

    for b in (0, 1):
        s = NSTEP - 2 + b
        bc, h0 = step_coords(s)
        bt = 4 * wid + bc
        pltpu.make_async_copy(
            tbs[b], out_hbm.at[pl.ds(h0, HB), :, bt], sss[b]
        ).wait()


@functools.partial(
    pl.kernel,
    mesh=plsc.VectorSubcoreMesh(core_axis_name="c", subcore_axis_name="s"),
    out_type=jax.ShapeDtypeStruct((HIST, EMBED_D // 8, BATCH // 128, 8, 128),
                                  jnp.float32),
    compiler_params=pltpu.CompilerParams(use_tc_tiling_on_sc=False,
                                         needs_layout_passes=False),
    scratch_types=[
        pltpu.VMEM((HIST, PW), jnp.int32),
        pltpu.VMEM((HB * BC, EMBED_D), jnp.float32),
        pltpu.VMEM((HB * BC, EMBED_D), jnp.float32),
        pltpu.VMEM((HB, EMBED_D // 8, 8, BC), jnp.float32),
        pltpu.VMEM((HB, EMBED_D // 8, 8, BC), jnp.float32),
        pltpu.SemaphoreType.DMA,
        pltpu.SemaphoreType.DMA,
        pltpu.SemaphoreType.DMA,
        pltpu.SemaphoreType.DMA,
    ],
)
def _sc_gather(xt_hbm, table_hbm, out_hbm,
               idx_t, raw0, raw1, tb0, tb1, sg0, sg1, ss0, ss1):
    _gather_body(xt_hbm, table_hbm, out_hbm,
                 idx_t, raw0, raw1, tb0, tb1, sg0, sg1, ss0, ss1)


@jax.jit
def kernel(x, table):
    xt = x.reshape(NW, PW, HIST).astype(jnp.int32).transpose(0, 2, 1)
    out5 = _sc_gather(xt, table)
    return out5.transpose(2, 4, 0, 1, 3).reshape(BATCH, HIST, EMBED_D)


# speedup vs baseline: 1.8774x; 1.8774x over previous
import functools

import jax
import jax.numpy as jnp
from jax import lax
from jax.experimental import pallas as pl
from jax.experimental.pallas import tpu as pltpu
from jax.experimental.pallas import tpu_sc as plsc

BATCH = 16384
HIST = 50
EMBED_D = 32
NC, NS = 2, 16
NW = NC * NS
PW = BATCH // NW
HB = 5
BC = 128
NSTEP = (PW // BC) * (HIST // HB)


def _gather_body(xt_hbm, table_hbm, out_hbm,
                 idx_t, raw0, raw1, tb0, tb1, sg0, sg1, ss0, ss1):
    wid = lax.axis_index("s") * NC + lax.axis_index("c")
    raws = (raw0, raw1)
    tbs = (tb0, tb1)
    sgs = (sg0, sg1)
    sss = (ss0, ss1)

    pltpu.sync_copy(xt_hbm.at[wid], idx_t)

    iota = jax.lax.broadcasted_iota(jnp.int32, (16,), 0)
    dtv0 = iota // 8
    drv0 = iota % 8
    dtv1 = (iota + 16) // 8
    drv1 = (iota + 16) % 8
    hsps = [jnp.full((16,), hh, jnp.int32) for hh in range(HB)]

    def step_coords(s):
        bc = s // (HIST // HB)
        h0 = (s % (HIST // HB)) * HB
        return bc, h0

    def issue(s, b):
        bc, h0 = step_coords(s)
        for hh in range(HB):
            pltpu.async_copy(
                table_hbm.at[idx_t.at[h0 + hh, pl.ds(bc * BC, BC)]],
                raws[b].at[pl.ds(hh * BC, BC)],
                sgs[b],
            )

    def drain(s, b):
        bc, h0 = step_coords(s)
        for hh in range(HB):
            pltpu.make_async_copy(
                table_hbm.at[idx_t.at[h0 + hh, pl.ds(bc * BC, BC)]],
                raws[b].at[pl.ds(hh * BC, BC)],
                sgs[b],
            ).wait()

    issue(0, 0)
    issue(1, 1)

    def body(i, carry):
        for b in (0, 1):
            s = 2 * i + b
            bc, h0 = step_coords(s)
            bt = 4 * wid + bc
            drain(s, b)

            @pl.when(s >= 2)
            def _():
                pltpu.make_async_copy(
                    tbs[b].at[:, :, :, pl.ds(0, BC)],
                    out_hbm.at[pl.ds(h0, HB), :, bt], sss[b]
                ).wait()

            def bloop(bb, c):
                for u in (0, 1):
                    bbu = 2 * bb + u
                    bsp = jnp.full((16,), bbu, jnp.int32)
                    for hh in range(HB):
                        row = hh * BC + bbu
                        hsp = hsps[hh]
                        v0 = raws[b][row, pl.ds(0, 16)]
                        v1 = raws[b][row, pl.ds(16, 16)]
                        plsc.store_scatter(tbs[b], [hsp, dtv0, drv0, bsp], v0)
                        plsc.store_scatter(tbs[b], [hsp, dtv1, drv1, bsp], v1)
                return c

            lax.fori_loop(0, BC // 2, bloop, 0)

            pltpu.async_copy(tbs[b].at[:, :, :, pl.ds(0, BC)],
                             out_hbm.at[pl.ds(h0, HB), :, bt], sss[b])

            nxt = s + 2

            @pl.when(nxt < NSTEP)
            def _():
                issue(nxt, b)

        return carry

    lax.fori_loop(0, NSTEP // 2, body, 0)

    for b in (0, 1):
        s = NSTEP - 2 + b
        bc, h0 = step_coords(s)
        bt = 4 * wid + bc
        pltpu.make_async_copy(
            tbs[b].at[:, :, :, pl.ds(0, BC)],
            out_hbm.at[pl.ds(h0, HB), :, bt], sss[b]
        ).wait()


@functools.partial(
    pl.kernel,
    mesh=plsc.VectorSubcoreMesh(core_axis_name="c", subcore_axis_name="s"),
    out_type=jax.ShapeDtypeStruct((HIST, EMBED_D // 8, BATCH // 128, 8, 128),
                                  jnp.float32),
    compiler_params=pltpu.CompilerParams(use_tc_tiling_on_sc=False,
                                         needs_layout_passes=False),
    scratch_types=[
        pltpu.VMEM((HIST, PW), jnp.int32),
        pltpu.VMEM((HB * BC, EMBED_D), jnp.float32),
        pltpu.VMEM((HB * BC, EMBED_D), jnp.float32),
        pltpu.VMEM((HB, EMBED_D // 8, 8, BC + 1), jnp.float32),
        pltpu.VMEM((HB, EMBED_D // 8, 8, BC + 1), jnp.float32),
        pltpu.SemaphoreType.DMA,
        pltpu.SemaphoreType.DMA,
        pltpu.SemaphoreType.DMA,
        pltpu.SemaphoreType.DMA,
    ],
)
def _sc_gather(xt_hbm, table_hbm, out_hbm,
               idx_t, raw0, raw1, tb0, tb1, sg0, sg1, ss0, ss1):
    _gather_body(xt_hbm, table_hbm, out_hbm,
                 idx_t, raw0, raw1, tb0, tb1, sg0, sg1, ss0, ss1)


@jax.jit
def kernel(x, table):
    xt = x.reshape(NW, PW, HIST).astype(jnp.int32).transpose(0, 2, 1)
    out5 = _sc_gather(xt, table)
    return out5.transpose(2, 4, 0, 1, 3).reshape(BATCH, HIST, EMBED_D)


# speedup vs baseline: 1.8846x; 1.0039x over previous
import functools

import jax
import jax.numpy as jnp
from jax import lax
from jax.experimental import pallas as pl
from jax.experimental.pallas import tpu as pltpu
from jax.experimental.pallas import tpu_sc as plsc

BATCH = 16384
HIST = 50
EMBED_D = 32
NC, NS = 2, 16
NW = NC * NS
PW = BATCH // NW
HB = 5
BC = 128
NSTEP = (PW // BC) * (HIST // HB)


def _gather_body(xt_hbm, table_hbm, out_hbm,
                 idx_t, raw0, raw1, tb0, tb1, sg0, sg1, ss0, ss1):
    wid = lax.axis_index("s") * NC + lax.axis_index("c")
    raws = (raw0, raw1)
    tbs = (tb0, tb1)
    sgs = (sg0, sg1)
    sss = (ss0, ss1)

    pltpu.sync_copy(xt_hbm.at[wid], idx_t)

    iota = jax.lax.broadcasted_iota(jnp.int32, (16,), 0)
    dtv0 = iota // 8
    drv0 = iota % 8
    dtv1 = (iota + 16) // 8
    drv1 = (iota + 16) % 8
    hsps = [jnp.full((16,), hh, jnp.int32) for hh in range(HB)]

    def step_coords(s):
        bc = s // (HIST // HB)
        h0 = (s % (HIST // HB)) * HB
        return bc, h0

    def issue(s, b):
        bc, h0 = step_coords(s)
        for hh in range(HB):
            pltpu.async_copy(
                table_hbm.at[idx_t.at[h0 + hh, pl.ds(bc * BC, BC)]],
                raws[b].at[pl.ds(hh * BC, BC)],
                sgs[b],
            )

    def drain(s, b):
        bc, h0 = step_coords(s)
        for hh in range(HB):
            pltpu.make_async_copy(
                table_hbm.at[idx_t.at[h0 + hh, pl.ds(bc * BC, BC)]],
                raws[b].at[pl.ds(hh * BC, BC)],
                sgs[b],
            ).wait()

    issue(0, 0)
    issue(1, 1)

    def body(i, carry):
        for b in (0, 1):
            s = 2 * i + b
            bc, h0 = step_coords(s)
            bt = 4 * wid + bc
            drain(s, b)

            @pl.when(s >= 2)
            def _():
                pltpu.make_async_copy(
                    tbs[b].at[:, :, :, pl.ds(0, BC)],
                    out_hbm.at[pl.ds(h0, HB), :, bt], sss[b]
                ).wait()

            def bloop(bb, c):
                for u in (0, 1, 2, 3):
                    bbu = 4 * bb + u
                    bsp = jnp.full((16,), bbu, jnp.int32)
                    for hh in range(HB):
                        row = hh * BC + bbu
                        hsp = hsps[hh]
                        v0 = raws[b][row, pl.ds(0, 16)]
                        v1 = raws[b][row, pl.ds(16, 16)]
                        plsc.store_scatter(tbs[b], [hsp, dtv0, drv0, bsp], v0)
                        plsc.store_scatter(tbs[b], [hsp, dtv1, drv1, bsp], v1)
                return c

            lax.fori_loop(0, BC // 4, bloop, 0)

            pltpu.async_copy(tbs[b].at[:, :, :, pl.ds(0, BC)],
                             out_hbm.at[pl.ds(h0, HB), :, bt], sss[b])

            nxt = s + 2

            @pl.when(nxt < NSTEP)
            def _():
                issue(nxt, b)

        return carry

    lax.fori_loop(0, NSTEP // 2, body, 0)

    for b in (0, 1):
        s = NSTEP - 2 + b
        bc, h0 = step_coords(s)
        bt = 4 * wid + bc
        pltpu.make_async_copy(
            tbs[b].at[:, :, :, pl.ds(0, BC)],
            out_hbm.at[pl.ds(h0, HB), :, bt], sss[b]
        ).wait()


@functools.partial(
    pl.kernel,
    mesh=plsc.VectorSubcoreMesh(core_axis_name="c", subcore_axis_name="s"),
    out_type=jax.ShapeDtypeStruct((HIST, EMBED_D // 8, BATCH // 128, 8, 128),
                                  jnp.float32),
    compiler_params=pltpu.CompilerParams(use_tc_tiling_on_sc=False,
                                         needs_layout_passes=False),
    scratch_types=[
        pltpu.VMEM((HIST, PW), jnp.int32),
        pltpu.VMEM((HB * BC, EMBED_D), jnp.float32),
        pltpu.VMEM((HB * BC, EMBED_D), jnp.float32),
        pltpu.VMEM((HB, EMBED_D // 8, 8, BC + 1), jnp.float32),
        pltpu.VMEM((HB, EMBED_D // 8, 8, BC + 1), jnp.float32),
        pltpu.SemaphoreType.DMA,
        pltpu.SemaphoreType.DMA,
        pltpu.SemaphoreType.DMA,
        pltpu.SemaphoreType.DMA,
    ],
)
def _sc_gather(xt_hbm, table_hbm, out_hbm,
               idx_t, raw0, raw1, tb0, tb1, sg0, sg1, ss0, ss1):
    _gather_body(xt_hbm, table_hbm, out_hbm,
                 idx_t, raw0, raw1, tb0, tb1, sg0, sg1, ss0, ss1)


@jax.jit
def kernel(x, table):
    xt = x.reshape(NW, PW, HIST).astype(jnp.int32).transpose(0, 2, 1)
    out5 = _sc_gather(xt, table)
    return out5.transpose(2, 4, 0, 1, 3).reshape(BATCH, HIST, EMBED_D)
